# baseline (speedup 1.0000x reference)
import numpy as np

import jax
import jax.numpy as jnp
from jax.experimental import pallas as pl
from jax.experimental.pallas import tpu as pltpu


def _dist2_threshold() -> np.float32:
    c = np.float32(0.03)
    t = np.float32(np.float64(c) * np.float64(c))
    while np.sqrt(t) >= c:
        t = np.nextafter(t, np.float32(0.0), dtype=np.float32)
    while np.sqrt(t) < c:
        t = np.nextafter(t, np.float32(np.inf), dtype=np.float32)
    return t


DIST2_THRESH = float(_dist2_threshold())


def _body(rows_ref, emba_ref, cols_ref, embc_ref, onesd_ref, W1_ref, b1_ref,
          W2_ref, b2_ref, out_ref, embn_ref):
    i = pl.program_id(1)
    TI = embc_ref.shape[1]
    rows = rows_ref[0]
    emba = emba_ref[0]
    cols = cols_ref[0]
    embc = embc_ref[0]

    @pl.when(i == 0)
    def _build_normalized():
        emb = emba[:, 0:64]
        sq = emb * emb
        nr2 = jax.lax.dot_general(
            sq, onesd_ref[...], (((1,), (0,)), ((), ())))
        inv = 1.0 / jnp.maximum(jnp.sqrt(nr2), 1e-8)
        embn_ref[...] = emb * inv

    px_row = rows[0:1, :]
    py_row = rows[1:2, :]
    t_row = rows[2:3, :]
    px_col = cols[:, 0:1]
    py_col = cols[:, 1:2]
    mask_col = cols[:, 2:3] > 0.0

    dx = px_row - px_col
    dy = py_row - py_col
    d2 = dx * dx + dy * dy
    neighbor = d2 < t_row
    neighbor_f = neighbor.astype(jnp.float32)
    n_count = jax.lax.dot_general(
        neighbor_f, emba[:, 64:65], (((1,), (0,)), ((), ())))

    embn_c = embn_ref[pl.ds(i * TI, TI), :]
    gram_s = jax.lax.dot_general(
        embn_c, embn_ref[...], (((1,), (1,)), ((), ())))
    similar_f = (neighbor & (gram_s > 0.7)).astype(jnp.float32)

    acc = jax.lax.dot_general(
        similar_f, emba, (((1,), (0,)), ((), ())))
    cnt_sim = acc[:, 64:65]
    mean_sim = acc[:, 0:64] / jnp.maximum(cnt_sim, 1.0)

    combined = jnp.concatenate([embc, mean_sim], axis=1)
    h = jnp.maximum(combined @ W1_ref[...] + b1_ref[...], 0.0)
    out = h @ W2_ref[...] + b2_ref[...]

    update = mask_col & (n_count > 1.0) & (cnt_sim > 0.0)
    refined = jnp.where(update, out, embc)
    leaf_count = jnp.sum(rows[3:4, :], axis=1, keepdims=True)
    out_ref[0] = jnp.where(leaf_count < 10.0, embc, refined)


@jax.jit
def kernel(points, embeddings, leaf_mask, W1, b1, W2, b2):
    B, N, D = embeddings.shape
    TI = 256

    mask_f = leaf_mask.astype(jnp.float32)
    t_masked = jnp.where(leaf_mask > 0, jnp.float32(DIST2_THRESH),
                         jnp.float32(-1.0))
    rows = jnp.concatenate(
        [jnp.transpose(points, (0, 2, 1)), t_masked[:, None, :],
         mask_f[:, None, :], jnp.zeros((B, 4, N), jnp.float32)], axis=1)
    cols = jnp.concatenate(
        [points, mask_f[:, :, None], jnp.zeros((B, N, 5), jnp.float32)],
        axis=2)
    emba = jnp.concatenate(
        [embeddings, jnp.ones((B, N, 1), jnp.float32),
         jnp.zeros((B, N, 7), jnp.float32)], axis=2)
    onesd = jnp.ones((D, 1), jnp.float32)

    grid = (B, N // TI)
    return pl.pallas_call(
        _body,
        grid=grid,
        in_specs=[
            pl.BlockSpec((1, 8, N), lambda b, i: (b, 0, 0)),
            pl.BlockSpec((1, N, 72), lambda b, i: (b, 0, 0)),
            pl.BlockSpec((1, TI, 8), lambda b, i: (b, i, 0)),
            pl.BlockSpec((1, TI, D), lambda b, i: (b, i, 0)),
            pl.BlockSpec((D, 1), lambda b, i: (0, 0)),
            pl.BlockSpec((2 * D, D), lambda b, i: (0, 0)),
            pl.BlockSpec((1, D), lambda b, i: (0, 0)),
            pl.BlockSpec((D, D), lambda b, i: (0, 0)),
            pl.BlockSpec((1, D), lambda b, i: (0, 0)),
        ],
        out_specs=pl.BlockSpec((1, TI, D), lambda b, i: (b, i, 0)),
        out_shape=jax.ShapeDtypeStruct((B, N, D), jnp.float32),
        scratch_shapes=[pltpu.VMEM((N, D), jnp.float32)],
    )(rows, emba, cols, embeddings, onesd, W1, b1.reshape(1, D), W2,
      b2.reshape(1, D))


# speedup vs baseline: 1.1508x; 1.1508x over previous
import numpy as np

import jax
import jax.numpy as jnp
from jax.experimental import pallas as pl
from jax.experimental.pallas import tpu as pltpu


def _dist2_threshold() -> np.float32:
    c = np.float32(0.03)
    t = np.float32(np.float64(c) * np.float64(c))
    while np.sqrt(t) >= c:
        t = np.nextafter(t, np.float32(0.0), dtype=np.float32)
    while np.sqrt(t) < c:
        t = np.nextafter(t, np.float32(np.inf), dtype=np.float32)
    return t


DIST2_THRESH = float(_dist2_threshold())


def _body(rows_ref, embx_ref, cols_ref, embc_ref, W1_ref, b1_ref, W2_ref,
          b2_ref, out_ref, embn_ref):
    i = pl.program_id(1)
    rows = rows_ref[0]
    embx = embx_ref[0]
    cols = cols_ref[0]
    embc = embc_ref[0]

    @pl.when(i == 0)
    def _build_normalized():
        emb_t = embx[0:64, :]
        nr2 = jnp.sum(emb_t * emb_t, axis=0, keepdims=True)
        inv = 1.0 / jnp.maximum(jnp.sqrt(nr2), 1e-8)
        embn_ref[...] = emb_t * inv

    px_row = rows[0:1, :]
    py_row = rows[1:2, :]
    t_row = rows[2:3, :]
    px_col = cols[:, 0:1]
    py_col = cols[:, 1:2]
    mask_col = cols[:, 2:3] > 0.0

    dx = px_row - px_col
    dy = py_row - py_col
    d2 = dx * dx + dy * dy
    neighbor = d2 < t_row
    neighbor_f = neighbor.astype(jnp.float32)
    n_count = jax.lax.dot_general(
        neighbor_f, embx[64:65, :], (((1,), (1,)), ((), ())))

    nc2 = jnp.sum(embc * embc, axis=1, keepdims=True)
    embc_n = embc * (1.0 / jnp.maximum(jnp.sqrt(nc2), 1e-8))
    gram_s = jax.lax.dot_general(
        embc_n, embn_ref[...], (((1,), (0,)), ((), ())))
    similar_f = (neighbor & (gram_s > 0.7)).astype(jnp.float32)

    acc = jax.lax.dot_general(
        similar_f, embx, (((1,), (1,)), ((), ())))
    cnt_sim = acc[:, 64:65]
    mean_sim = acc[:, 0:64] / jnp.maximum(cnt_sim, 1.0)

    combined = jnp.concatenate([embc, mean_sim], axis=1)
    h = jnp.maximum(combined @ W1_ref[...] + b1_ref[...], 0.0)
    out = h @ W2_ref[...] + b2_ref[...]

    update = mask_col & (n_count > 1.0) & (cnt_sim > 0.0)
    refined = jnp.where(update, out, embc)
    leaf_count = jnp.sum(rows[3:4, :], axis=1, keepdims=True)
    out_ref[0] = jnp.where(leaf_count < 10.0, embc, refined)


@jax.jit
def kernel(points, embeddings, leaf_mask, W1, b1, W2, b2):
    B, N, D = embeddings.shape
    TI = 256

    mask_f = leaf_mask.astype(jnp.float32)
    t_masked = jnp.where(leaf_mask > 0, jnp.float32(DIST2_THRESH),
                         jnp.float32(-1.0))
    rows = jnp.concatenate(
        [jnp.transpose(points, (0, 2, 1)), t_masked[:, None, :],
         mask_f[:, None, :], jnp.zeros((B, 4, N), jnp.float32)], axis=1)
    cols = jnp.concatenate(
        [points, mask_f[:, :, None], jnp.zeros((B, N, 5), jnp.float32)],
        axis=2)
    embx = jnp.concatenate(
        [jnp.transpose(embeddings, (0, 2, 1)),
         jnp.ones((B, 1, N), jnp.float32),
         jnp.zeros((B, 7, N), jnp.float32)], axis=1)

    grid = (B, N // TI)
    return pl.pallas_call(
        _body,
        grid=grid,
        in_specs=[
            pl.BlockSpec((1, 8, N), lambda b, i: (b, 0, 0)),
            pl.BlockSpec((1, 72, N), lambda b, i: (b, 0, 0)),
            pl.BlockSpec((1, TI, 8), lambda b, i: (b, i, 0)),
            pl.BlockSpec((1, TI, D), lambda b, i: (b, i, 0)),
            pl.BlockSpec((2 * D, D), lambda b, i: (0, 0)),
            pl.BlockSpec((1, D), lambda b, i: (0, 0)),
            pl.BlockSpec((D, D), lambda b, i: (0, 0)),
            pl.BlockSpec((1, D), lambda b, i: (0, 0)),
        ],
        out_specs=pl.BlockSpec((1, TI, D), lambda b, i: (b, i, 0)),
        out_shape=jax.ShapeDtypeStruct((B, N, D), jnp.float32),
        scratch_shapes=[pltpu.VMEM((D, N), jnp.float32)],
    )(rows, embx, cols, embeddings, W1, b1.reshape(1, D), W2,
      b2.reshape(1, D))


# speedup vs baseline: 1.1706x; 1.0171x over previous
import numpy as np

import jax
import jax.numpy as jnp
from jax.experimental import pallas as pl
from jax.experimental.pallas import tpu as pltpu


def _dist2_threshold() -> np.float32:
    c = np.float32(0.03)
    t = np.float32(np.float64(c) * np.float64(c))
    while np.sqrt(t) >= c:
        t = np.nextafter(t, np.float32(0.0), dtype=np.float32)
    while np.sqrt(t) < c:
        t = np.nextafter(t, np.float32(np.inf), dtype=np.float32)
    return t


DIST2_THRESH = float(_dist2_threshold())


def _body(rows_ref, embx_ref, cols_ref, embc_ref, W1_ref, b1_ref, W2_ref,
          b2_ref, out_ref, embn_ref):
    i = pl.program_id(1)
    rows = rows_ref[0]
    embx = embx_ref[0]
    cols = cols_ref[0]
    embc = embc_ref[0]

    @pl.when(i == 0)
    def _build_normalized():
        emb_t = embx[0:64, :]
        nr2 = jnp.sum(emb_t * emb_t, axis=0, keepdims=True)
        inv = 1.0 / jnp.maximum(jnp.sqrt(nr2), 1e-8)
        embn_ref[...] = emb_t * inv

    px_row = rows[0:1, :]
    py_row = rows[1:2, :]
    t_row = rows[2:3, :]
    px_col = cols[:, 0:1]
    py_col = cols[:, 1:2]
    mask_col = cols[:, 2:3] > 0.0

    dx = px_row - px_col
    dy = py_row - py_col
    d2 = dx * dx + dy * dy
    neighbor = d2 < t_row
    neighbor_f = neighbor.astype(jnp.float32)
    n_count = jax.lax.dot_general(
        neighbor_f, embx[64:65, :], (((1,), (1,)), ((), ())))

    nc2 = jnp.sum(embc * embc, axis=1, keepdims=True)
    embc_n = embc * (1.0 / jnp.maximum(jnp.sqrt(nc2), 1e-8))
    gram_s = jax.lax.dot_general(
        embc_n, embn_ref[...], (((1,), (0,)), ((), ())))
    similar_f = (neighbor & (gram_s > 0.7)).astype(jnp.float32)

    acc = jax.lax.dot_general(
        similar_f, embx, (((1,), (1,)), ((), ())))
    cnt_sim = acc[:, 64:65]
    mean_sim = acc[:, 0:64] / jnp.maximum(cnt_sim, 1.0)

    combined = jnp.concatenate([embc, mean_sim], axis=1)
    h = jnp.maximum(combined @ W1_ref[...] + b1_ref[...], 0.0)
    out = h @ W2_ref[...] + b2_ref[...]

    update = mask_col & (n_count > 1.0) & (cnt_sim > 0.0)
    refined = jnp.where(update, out, embc)
    leaf_count = jnp.sum(rows[3:4, :], axis=1, keepdims=True)
    out_ref[0] = jnp.where(leaf_count < 10.0, embc, refined)


@jax.jit
def kernel(points, embeddings, leaf_mask, W1, b1, W2, b2):
    B, N, D = embeddings.shape
    TI = 512

    mask_f = leaf_mask.astype(jnp.float32)
    t_masked = jnp.where(leaf_mask > 0, jnp.float32(DIST2_THRESH),
                         jnp.float32(-1.0))
    rows = jnp.concatenate(
        [jnp.transpose(points, (0, 2, 1)), t_masked[:, None, :],
         mask_f[:, None, :], jnp.zeros((B, 4, N), jnp.float32)], axis=1)
    cols = jnp.concatenate(
        [points, mask_f[:, :, None], jnp.zeros((B, N, 5), jnp.float32)],
        axis=2)
    embx = jnp.concatenate(
        [jnp.transpose(embeddings, (0, 2, 1)),
         jnp.ones((B, 1, N), jnp.float32),
         jnp.zeros((B, 7, N), jnp.float32)], axis=1)

    grid = (B, N // TI)
    return pl.pallas_call(
        _body,
        grid=grid,
        in_specs=[
            pl.BlockSpec((1, 8, N), lambda b, i: (b, 0, 0)),
            pl.BlockSpec((1, 72, N), lambda b, i: (b, 0, 0)),
            pl.BlockSpec((1, TI, 8), lambda b, i: (b, i, 0)),
            pl.BlockSpec((1, TI, D), lambda b, i: (b, i, 0)),
            pl.BlockSpec((2 * D, D), lambda b, i: (0, 0)),
            pl.BlockSpec((1, D), lambda b, i: (0, 0)),
            pl.BlockSpec((D, D), lambda b, i: (0, 0)),
            pl.BlockSpec((1, D), lambda b, i: (0, 0)),
        ],
        out_specs=pl.BlockSpec((1, TI, D), lambda b, i: (b, i, 0)),
        out_shape=jax.ShapeDtypeStruct((B, N, D), jnp.float32),
        scratch_shapes=[pltpu.VMEM((D, N), jnp.float32)],
    )(rows, embx, cols, embeddings, W1, b1.reshape(1, D), W2,
      b2.reshape(1, D))


# speedup vs baseline: 1.2515x; 1.0692x over previous
import numpy as np

import jax
import jax.numpy as jnp
from jax.experimental import pallas as pl
from jax.experimental.pallas import tpu as pltpu


def _dist2_threshold() -> np.float32:
    c = np.float32(0.03)
    t = np.float32(np.float64(c) * np.float64(c))
    while np.sqrt(t) >= c:
        t = np.nextafter(t, np.float32(0.0), dtype=np.float32)
    while np.sqrt(t) < c:
        t = np.nextafter(t, np.float32(np.inf), dtype=np.float32)
    return t


DIST2_THRESH = float(_dist2_threshold())


def _body(rows_ref, embx_ref, cols_ref, embc_ref, W1_ref, b1_ref, W2_ref,
          b2_ref, out_ref, embn_ref):
    i = pl.program_id(1)
    rows = rows_ref[0]
    embx = embx_ref[0]
    cols = cols_ref[0]
    embc = embc_ref[0]

    @pl.when(i == 0)
    def _build_normalized():
        emb_t = embx[0:64, :]
        nr2 = jnp.sum(emb_t * emb_t, axis=0, keepdims=True)
        inv = 1.0 / jnp.maximum(jnp.sqrt(nr2), 1e-8)
        embn_ref[...] = emb_t * inv

    px_row = rows[0:1, :]
    py_row = rows[1:2, :]
    t_row = rows[2:3, :]
    px_col = cols[:, 0:1]
    py_col = cols[:, 1:2]
    mask_col = cols[:, 2:3] > 0.0

    dx = px_row - px_col
    dy = py_row - py_col
    d2 = dx * dx + dy * dy
    neighbor = d2 < t_row
    neighbor_f = neighbor.astype(jnp.float32)
    n_count = jax.lax.dot_general(
        neighbor_f, embx[64:65, :], (((1,), (1,)), ((), ())))

    nc2 = jnp.sum(embc * embc, axis=1, keepdims=True)
    embc_n = embc * (1.0 / jnp.maximum(jnp.sqrt(nc2), 1e-8))
    gram_s = jax.lax.dot_general(
        embc_n, embn_ref[...], (((1,), (0,)), ((), ())))
    similar_f = (neighbor & (gram_s > 0.7)).astype(jnp.float32)

    acc = jax.lax.dot_general(
        similar_f, embx, (((1,), (1,)), ((), ())))
    cnt_sim = acc[:, 64:65]
    mean_sim = acc[:, 0:64] / jnp.maximum(cnt_sim, 1.0)

    combined = jnp.concatenate([embc, mean_sim], axis=1)
    h = jnp.maximum(combined @ W1_ref[...] + b1_ref[...], 0.0)
    out = h @ W2_ref[...] + b2_ref[...]

    update = mask_col & (n_count > 1.0) & (cnt_sim > 0.0)
    refined = jnp.where(update, out, embc)
    leaf_count = jnp.sum(rows[3:4, :], axis=1, keepdims=True)
    out_ref[0] = jnp.where(leaf_count < 10.0, embc, refined)


@jax.jit
def kernel(points, embeddings, leaf_mask, W1, b1, W2, b2):
    B, N, D = embeddings.shape
    TI = 1024

    mask_f = leaf_mask.astype(jnp.float32)
    t_masked = jnp.where(leaf_mask > 0, jnp.float32(DIST2_THRESH),
                         jnp.float32(-1.0))
    rows = jnp.concatenate(
        [jnp.transpose(points, (0, 2, 1)), t_masked[:, None, :],
         mask_f[:, None, :], jnp.zeros((B, 4, N), jnp.float32)], axis=1)
    cols = jnp.concatenate(
        [points, mask_f[:, :, None], jnp.zeros((B, N, 5), jnp.float32)],
        axis=2)
    embx = jnp.concatenate(
        [jnp.transpose(embeddings, (0, 2, 1)),
         jnp.ones((B, 1, N), jnp.float32),
         jnp.zeros((B, 7, N), jnp.float32)], axis=1)

    grid = (B, N // TI)
    return pl.pallas_call(
        _body,
        grid=grid,
        in_specs=[
            pl.BlockSpec((1, 8, N), lambda b, i: (b, 0, 0)),
            pl.BlockSpec((1, 72, N), lambda b, i: (b, 0, 0)),
            pl.BlockSpec((1, TI, 8), lambda b, i: (b, i, 0)),
            pl.BlockSpec((1, TI, D), lambda b, i: (b, i, 0)),
            pl.BlockSpec((2 * D, D), lambda b, i: (0, 0)),
            pl.BlockSpec((1, D), lambda b, i: (0, 0)),
            pl.BlockSpec((D, D), lambda b, i: (0, 0)),
            pl.BlockSpec((1, D), lambda b, i: (0, 0)),
        ],
        out_specs=pl.BlockSpec((1, TI, D), lambda b, i: (b, i, 0)),
        out_shape=jax.ShapeDtypeStruct((B, N, D), jnp.float32),
        scratch_shapes=[pltpu.VMEM((D, N), jnp.float32)],
    )(rows, embx, cols, embeddings, W1, b1.reshape(1, D), W2,
      b2.reshape(1, D))


# speedup vs baseline: 1.2889x; 1.0298x over previous
import numpy as np

import jax
import jax.numpy as jnp
from jax.experimental import pallas as pl
from jax.experimental.pallas import tpu as pltpu


def _dist2_threshold() -> np.float32:
    c = np.float32(0.03)
    t = np.float32(np.float64(c) * np.float64(c))
    while np.sqrt(t) >= c:
        t = np.nextafter(t, np.float32(0.0), dtype=np.float32)
    while np.sqrt(t) < c:
        t = np.nextafter(t, np.float32(np.inf), dtype=np.float32)
    return t


DIST2_THRESH = float(_dist2_threshold())


def _body(rows_ref, embx_ref, cols_ref, embc_ref, W1_ref, b1_ref, W2_ref,
          b2_ref, out_ref, embn_ref):
    i = pl.program_id(1)
    rows = rows_ref[0]
    embx = embx_ref[0]
    cols = cols_ref[0]
    embc = embc_ref[0]

    @pl.when(i == 0)
    def _build_normalized():
        emb_t = embx[0:64, :]
        nr2 = jnp.sum(emb_t * emb_t, axis=0, keepdims=True)
        inv = 1.0 / jnp.maximum(jnp.sqrt(nr2), 1e-8)
        embn_ref[...] = emb_t * inv

    px_row = rows[0:1, :]
    py_row = rows[1:2, :]
    t_row = rows[2:3, :]
    px_col = cols[:, 0:1]
    py_col = cols[:, 1:2]
    mask_col = cols[:, 2:3] > 0.0

    dx = px_row - px_col
    dy = py_row - py_col
    d2 = dx * dx + dy * dy
    neighbor = d2 < t_row
    neighbor_f = neighbor.astype(jnp.float32)
    n_count = jax.lax.dot_general(
        neighbor_f, embx[64:65, :], (((1,), (1,)), ((), ())))

    nc2 = jnp.sum(embc * embc, axis=1, keepdims=True)
    embc_n = embc * (1.0 / jnp.maximum(jnp.sqrt(nc2), 1e-8))
    gram_s = jax.lax.dot_general(
        embc_n, embn_ref[...], (((1,), (0,)), ((), ())))
    similar_f = (neighbor & (gram_s > 0.7)).astype(jnp.float32)

    acc = jax.lax.dot_general(
        similar_f, embx, (((1,), (1,)), ((), ())))
    cnt_sim = acc[:, 64:65]
    mean_sim = acc[:, 0:64] / jnp.maximum(cnt_sim, 1.0)

    combined = jnp.concatenate([embc, mean_sim], axis=1)
    h = jnp.maximum(combined @ W1_ref[...] + b1_ref[...], 0.0)
    out = h @ W2_ref[...] + b2_ref[...]

    update = mask_col & (n_count > 1.0) & (cnt_sim > 0.0)
    refined = jnp.where(update, out, embc)
    leaf_count = jnp.sum(rows[3:4, :], axis=1, keepdims=True)
    out_ref[0] = jnp.where(leaf_count < 10.0, embc, refined)


@jax.jit
def kernel(points, embeddings, leaf_mask, W1, b1, W2, b2):
    B, N, D = embeddings.shape
    TI = 2048

    mask_f = leaf_mask.astype(jnp.float32)
    t_masked = jnp.where(leaf_mask > 0, jnp.float32(DIST2_THRESH),
                         jnp.float32(-1.0))
    rows = jnp.concatenate(
        [jnp.transpose(points, (0, 2, 1)), t_masked[:, None, :],
         mask_f[:, None, :], jnp.zeros((B, 4, N), jnp.float32)], axis=1)
    cols = jnp.concatenate(
        [points, mask_f[:, :, None], jnp.zeros((B, N, 5), jnp.float32)],
        axis=2)
    embx = jnp.concatenate(
        [jnp.transpose(embeddings, (0, 2, 1)),
         jnp.ones((B, 1, N), jnp.float32),
         jnp.zeros((B, 7, N), jnp.float32)], axis=1)

    grid = (B, N // TI)
    return pl.pallas_call(
        _body,
        grid=grid,
        in_specs=[
            pl.BlockSpec((1, 8, N), lambda b, i: (b, 0, 0)),
            pl.BlockSpec((1, 72, N), lambda b, i: (b, 0, 0)),
            pl.BlockSpec((1, TI, 8), lambda b, i: (b, i, 0)),
            pl.BlockSpec((1, TI, D), lambda b, i: (b, i, 0)),
            pl.BlockSpec((2 * D, D), lambda b, i: (0, 0)),
            pl.BlockSpec((1, D), lambda b, i: (0, 0)),
            pl.BlockSpec((D, D), lambda b, i: (0, 0)),
            pl.BlockSpec((1, D), lambda b, i: (0, 0)),
        ],
        out_specs=pl.BlockSpec((1, TI, D), lambda b, i: (b, i, 0)),
        out_shape=jax.ShapeDtypeStruct((B, N, D), jnp.float32),
        scratch_shapes=[pltpu.VMEM((D, N), jnp.float32)],
    )(rows, embx, cols, embeddings, W1, b1.reshape(1, D), W2,
      b2.reshape(1, D))


# speedup vs baseline: 1.3093x; 1.0159x over previous
import numpy as np

import jax
import jax.numpy as jnp
from jax.experimental import pallas as pl


def _dist2_threshold() -> np.float32:
    c = np.float32(0.03)
    t = np.float32(np.float64(c) * np.float64(c))
    while np.sqrt(t) >= c:
        t = np.nextafter(t, np.float32(0.0), dtype=np.float32)
    while np.sqrt(t) < c:
        t = np.nextafter(t, np.float32(np.inf), dtype=np.float32)
    return t


DIST2_THRESH = float(_dist2_threshold())


def _body(cols_ref, emba_ref, W1_ref, b1_ref, W2_ref, b2_ref, out_ref):
    cols = cols_ref[0]
    emba = emba_ref[0]
    emb = emba[:, 0:64]

    rows = jnp.transpose(cols)
    px_row = rows[0:1, :]
    py_row = rows[1:2, :]
    mask_row_f = rows[2:3, :]
    t_row = rows[3:4, :]
    px_col = cols[:, 0:1]
    py_col = cols[:, 1:2]
    mask_col = cols[:, 2:3] > 0.0

    dx = px_row - px_col
    dy = py_row - py_col
    d2 = dx * dx + dy * dy
    neighbor = d2 < t_row
    neighbor_f = neighbor.astype(jnp.float32)
    n_count = jax.lax.dot_general(
        neighbor_f, emba[:, 64:65], (((1,), (0,)), ((), ())))

    nc2 = jnp.sum(emb * emb, axis=1, keepdims=True)
    emb_n = emb * (1.0 / jnp.maximum(jnp.sqrt(nc2), 1e-8))
    gram_s = jax.lax.dot_general(
        emb_n, jnp.transpose(emb_n), (((1,), (0,)), ((), ())))
    similar_f = (neighbor & (gram_s > 0.7)).astype(jnp.float32)

    acc = jax.lax.dot_general(
        similar_f, emba, (((1,), (0,)), ((), ())))
    cnt_sim = acc[:, 64:65]
    mean_sim = acc[:, 0:64] / jnp.maximum(cnt_sim, 1.0)

    combined = jnp.concatenate([emb, mean_sim], axis=1)
    h = jnp.maximum(combined @ W1_ref[...] + b1_ref[...], 0.0)
    out = h @ W2_ref[...] + b2_ref[...]

    update = mask_col & (n_count > 1.0) & (cnt_sim > 0.0)
    refined = jnp.where(update, out, emb)
    leaf_count = jnp.sum(mask_row_f, axis=1, keepdims=True)
    out_ref[0] = jnp.where(leaf_count < 10.0, emb, refined)


@jax.jit
def kernel(points, embeddings, leaf_mask, W1, b1, W2, b2):
    B, N, D = embeddings.shape

    mask_f = leaf_mask.astype(jnp.float32)
    t_masked = jnp.where(leaf_mask > 0, jnp.float32(DIST2_THRESH),
                         jnp.float32(-1.0))
    cols = jnp.concatenate(
        [points, mask_f[:, :, None], t_masked[:, :, None],
         jnp.zeros((B, N, 4), jnp.float32)], axis=2)
    emba = jnp.concatenate(
        [embeddings, jnp.ones((B, N, 1), jnp.float32),
         jnp.zeros((B, N, 7), jnp.float32)], axis=2)

    return pl.pallas_call(
        _body,
        grid=(B,),
        in_specs=[
            pl.BlockSpec((1, N, 8), lambda b: (b, 0, 0)),
            pl.BlockSpec((1, N, 72), lambda b: (b, 0, 0)),
            pl.BlockSpec((2 * D, D), lambda b: (0, 0)),
            pl.BlockSpec((1, D), lambda b: (0, 0)),
            pl.BlockSpec((D, D), lambda b: (0, 0)),
            pl.BlockSpec((1, D), lambda b: (0, 0)),
        ],
        out_specs=pl.BlockSpec((1, N, D), lambda b: (b, 0, 0)),
        out_shape=jax.ShapeDtypeStruct((B, N, D), jnp.float32),
    )(cols, emba, W1, b1.reshape(1, D), W2, b2.reshape(1, D))


# speedup vs baseline: 1.5318x; 1.1699x over previous
import numpy as np

import jax
import jax.numpy as jnp
from jax.experimental import pallas as pl


def _dist2_threshold() -> np.float32:
    c = np.float32(0.03)
    t = np.float32(np.float64(c) * np.float64(c))
    while np.sqrt(t) >= c:
        t = np.nextafter(t, np.float32(0.0), dtype=np.float32)
    while np.sqrt(t) < c:
        t = np.nextafter(t, np.float32(np.inf), dtype=np.float32)
    return t


DIST2_THRESH = float(_dist2_threshold())


def _body(pts_ref, mask_ref, emb_ref, W1_ref, b1_ref, W2_ref, b2_ref,
          out_ref):
    pts = pts_ref[0]
    mask_f = mask_ref[0]
    emb = emb_ref[0]
    N = emb.shape[0]

    t_col = jnp.where(mask_f > 0.0, jnp.float32(DIST2_THRESH),
                      jnp.float32(-1.0))
    staged = jnp.concatenate([pts, t_col, mask_f], axis=1)
    rows = jnp.transpose(staged)
    px_row = rows[0:1, :]
    py_row = rows[1:2, :]
    t_row = rows[2:3, :]
    mask_row_f = rows[3:4, :]
    px_col = pts[:, 0:1]
    py_col = pts[:, 1:2]
    mask_col = mask_f > 0.0
    ones_col = jnp.ones((N, 1), jnp.float32)

    dx = px_row - px_col
    dy = py_row - py_col
    d2 = dx * dx + dy * dy
    neighbor = d2 < t_row
    neighbor_f = neighbor.astype(jnp.float32)
    n_count = jax.lax.dot_general(
        neighbor_f, ones_col, (((1,), (0,)), ((), ())))

    nc2 = jnp.sum(emb * emb, axis=1, keepdims=True)
    emb_n = emb * (1.0 / jnp.maximum(jnp.sqrt(nc2), 1e-8))
    gram_s = jax.lax.dot_general(
        emb_n, jnp.transpose(emb_n), (((1,), (0,)), ((), ())))
    similar_f = (neighbor & (gram_s > 0.7)).astype(jnp.float32)

    emb_aug = jnp.concatenate([emb, ones_col], axis=1)
    acc = jax.lax.dot_general(
        similar_f, emb_aug, (((1,), (0,)), ((), ())))
    cnt_sim = acc[:, 64:65]
    mean_sim = acc[:, 0:64] / jnp.maximum(cnt_sim, 1.0)

    combined = jnp.concatenate([emb, mean_sim], axis=1)
    h = jnp.maximum(combined @ W1_ref[...] + b1_ref[...], 0.0)
    out = h @ W2_ref[...] + b2_ref[...]

    update = mask_col & (n_count > 1.0) & (cnt_sim > 0.0)
    refined = jnp.where(update, out, emb)
    leaf_count = jnp.sum(mask_row_f, axis=1, keepdims=True)
    out_ref[0] = jnp.where(leaf_count < 10.0, emb, refined)


@jax.jit
def kernel(points, embeddings, leaf_mask, W1, b1, W2, b2):
    B, N, D = embeddings.shape
    mask_f = leaf_mask.astype(jnp.float32).reshape(B, N, 1)

    return pl.pallas_call(
        _body,
        grid=(B,),
        in_specs=[
            pl.BlockSpec((1, N, 2), lambda b: (b, 0, 0)),
            pl.BlockSpec((1, N, 1), lambda b: (b, 0, 0)),
            pl.BlockSpec((1, N, D), lambda b: (b, 0, 0)),
            pl.BlockSpec((2 * D, D), lambda b: (0, 0)),
            pl.BlockSpec((1, D), lambda b: (0, 0)),
            pl.BlockSpec((D, D), lambda b: (0, 0)),
            pl.BlockSpec((1, D), lambda b: (0, 0)),
        ],
        out_specs=pl.BlockSpec((1, N, D), lambda b: (b, 0, 0)),
        out_shape=jax.ShapeDtypeStruct((B, N, D), jnp.float32),
    )(points, mask_f, embeddings, W1, b1.reshape(1, D), W2,
      b2.reshape(1, D))
